# baseline (speedup 1.0000x reference)
import functools

import jax
import jax.numpy as jnp
from jax import lax
from jax.experimental import pallas as pl
from jax.experimental.pallas import tpu as pltpu
from jax.experimental.pallas import tpu_sc as plsc

_K = 8
_E = 64
_LANES = 16


@functools.lru_cache(maxsize=1)
def _noise_const(B, L, E):
    n = jax.random.normal(jax.random.key(1), (B, L, E), jnp.float32)
    return n.reshape(B * L, E)


def _logits_body(x_ref, wt_ref, b_ref, noise_ref, noisy_ref):
    z = jnp.dot(x_ref[...], wt_ref[...], preferred_element_type=jnp.float32)
    z = z + b_ref[...]
    logits = z[:, :_E]
    noise_logits = z[:, _E:]
    sp = jnp.maximum(noise_logits, 0.0) + jnp.log1p(jnp.exp(-jnp.abs(noise_logits)))
    noisy_ref[...] = logits + noise_ref[...] * sp


def _route_body(tpw, noisy_flat, outp, idxT, stage_v, vals_v, outp_v, idx_v, sem):
    wid = lax.axis_index("s") * 2 + lax.axis_index("c")
    base = wid * tpw
    cp_in = pltpu.async_copy(
        noisy_flat.at[pl.ds(base * _E, tpw * _E)], stage_v, sem)

    zero16 = jnp.zeros((_LANES,), jnp.float32)

    def zbody(i, c):
        outp_v[pl.ds(pl.multiple_of(i * _LANES, _LANES), _LANES)] = zero16
        return c

    lax.fori_loop(0, tpw * _E // _LANES, zbody, 0)
    cp_in.wait()

    lane = lax.broadcasted_iota(jnp.int32, (_LANES,), 0)
    neg_inf = jnp.full((_LANES,), -jnp.inf, jnp.float32)

    def tbody(g, c):
        tokv = g * _LANES + lane
        for j in range(_E // _LANES):
            for d in range(_LANES):
                e_lane = j * _LANES + ((lane + d) & (_LANES - 1))
                v = plsc.load_gather(stage_v, [tokv * _E + e_lane])
                plsc.store_scatter(vals_v, [e_lane * tpw + tokv], v)
        return c

    lax.fori_loop(0, tpw // _LANES, tbody, 0)

    def gbody(g, c):
        col0 = pl.multiple_of(g * _LANES, _LANES)
        tok = g * _LANES + lane
        ms, mis = [], []
        for _ in range(_K):
            def ebody(e, carry):
                m, mi = carry
                v = vals_v[pl.ds(e * tpw + col0, _LANES)]
                better = v > m
                return (jnp.where(better, v, m),
                        jnp.where(better, jnp.full((_LANES,), e, jnp.int32), mi))

            m, mi = lax.fori_loop(
                0, _E, ebody,
                (neg_inf, jnp.zeros((_LANES,), jnp.int32)), unroll=8)
            plsc.store_scatter(vals_v, [mi * tpw + tok], neg_inf)
            ms.append(m)
            mis.append(mi)

        m0 = ms[0]
        ws = [jnp.exp(m - m0) for m in ms]
        denom = ws[0]
        for w in ws[1:]:
            denom = denom + w
        inv = 1.0 / denom
        for k in range(_K):
            plsc.store_scatter(outp_v, [tok * _E + mis[k]], ws[k] * inv)
            idx_v[pl.ds(k * tpw + col0, _LANES)] = mis[k]
        return c

    lax.fori_loop(0, tpw // _LANES, gbody, 0)

    pltpu.sync_copy(outp_v, outp.at[pl.ds(base * _E, tpw * _E)])
    cps = [
        pltpu.async_copy(
            idx_v.at[pl.ds(k * tpw, tpw)],
            idxT.at[k, pl.ds(base, tpw)], sem)
        for k in range(_K)
    ]
    for cp in cps:
        cp.wait()


def kernel(x_BLD, W_route, b_route, W_noise, b_noise):
    B, L, D = x_BLD.shape
    E = W_route.shape[0]
    N = B * L
    T = 1024
    assert N % T == 0 and E == _E

    info = plsc.get_sparse_core_info()
    nw = info.num_cores * info.num_subcores
    tpw = N // nw
    spb = T // tpw

    x = x_BLD.reshape(N, D)
    wt = jnp.concatenate([W_route, W_noise], axis=0).T
    b = jnp.concatenate([b_route, b_noise]).reshape(1, 2 * E)
    noise = _noise_const(B, L, E)

    noisy = pl.pallas_call(
        _logits_body,
        grid=(N // T,),
        in_specs=[
            pl.BlockSpec((T, D), lambda i: (i, 0)),
            pl.BlockSpec((D, 2 * E), lambda i: (0, 0)),
            pl.BlockSpec((1, 2 * E), lambda i: (0, 0)),
            pl.BlockSpec((T, E), lambda i: (i, 0)),
        ],
        out_specs=pl.BlockSpec((T, E), lambda i: (i, 0)),
        out_shape=jax.ShapeDtypeStruct((N, E), jnp.float32),
        compiler_params=pltpu.CompilerParams(
            dimension_semantics=("arbitrary",),
        ),
    )(x, wt, b, noise)

    route = functools.partial(
        pl.kernel,
        out_type=[
            jax.ShapeDtypeStruct((N * E,), jnp.float32),
            jax.ShapeDtypeStruct((_K, N), jnp.int32),
        ],
        scratch_types=[
            pltpu.VMEM((tpw * E,), jnp.float32),
            pltpu.VMEM((E * tpw,), jnp.float32),
            pltpu.VMEM((tpw * E,), jnp.float32),
            pltpu.VMEM((_K * tpw,), jnp.int32),
            pltpu.SemaphoreType.DMA,
        ],
        mesh=plsc.VectorSubcoreMesh(core_axis_name="c", subcore_axis_name="s"),
        compiler_params=pltpu.CompilerParams(needs_layout_passes=False),
    )(functools.partial(_route_body, tpw))

    outp, idxT = route(noisy.reshape(N * E))
    return outp.reshape(B, L, E), idxT.T.reshape(B, L, _K)


# speedup vs baseline: 1.2220x; 1.2220x over previous
import functools

import jax
import jax.numpy as jnp
from jax import lax
from jax.experimental import pallas as pl
from jax.experimental.pallas import tpu as pltpu
from jax.experimental.pallas import tpu_sc as plsc

_K = 8
_E = 64
_LANES = 16


@functools.lru_cache(maxsize=1)
def _noise_const_slabs(B, L, E, tpw):
    n = jax.random.normal(jax.random.key(1), (B, L, E), jnp.float32)
    N = B * L
    return n.reshape(N // tpw, tpw, E).transpose(0, 2, 1)


def _logits_body(tpw, x_ref, wt_ref, b_ref, noiseT_ref, noisyT_ref):
    z = jnp.dot(x_ref[...], wt_ref[...], preferred_element_type=jnp.float32)
    z = z + b_ref[...]
    zT = z.T
    logits = zT[:_E, :]
    noise_logits = zT[_E:, :]
    sp = jnp.maximum(noise_logits, 0.0) + jnp.log1p(jnp.exp(-jnp.abs(noise_logits)))
    noisyT = logits
    for i in range(noisyT.shape[1] // tpw):
        sl = slice(i * tpw, (i + 1) * tpw)
        noisyT_ref[i] = noisyT[:, sl] + noiseT_ref[i] * sp[:, sl]


def _route_body(tpw, noisyT_flat, outp, idxT, vals_v, outp_v, idx_v, sem):
    wid = lax.axis_index("s") * 2 + lax.axis_index("c")
    base = wid * tpw
    cp_in = pltpu.async_copy(
        noisyT_flat.at[pl.ds(wid * _E * tpw, _E * tpw)], vals_v, sem)

    zero16 = jnp.zeros((_LANES,), jnp.float32)

    def zbody(i, c):
        outp_v[pl.ds(pl.multiple_of(i * _LANES, _LANES), _LANES)] = zero16
        return c

    lax.fori_loop(0, tpw * _E // _LANES, zbody, 0)
    cp_in.wait()

    lane = lax.broadcasted_iota(jnp.int32, (_LANES,), 0)
    neg_inf = jnp.full((_LANES,), -jnp.inf, jnp.float32)

    def gbody(g, c):
        col0 = pl.multiple_of(g * _LANES, _LANES)
        tok = g * _LANES + lane
        ms, mis = [], []
        for _ in range(_K):
            def ebody(e, carry):
                m, mi = carry
                v = vals_v[pl.ds(e * tpw + col0, _LANES)]
                better = v > m
                return (jnp.where(better, v, m),
                        jnp.where(better, jnp.full((_LANES,), e, jnp.int32), mi))

            m, mi = lax.fori_loop(
                0, _E, ebody,
                (neg_inf, jnp.zeros((_LANES,), jnp.int32)), unroll=8)
            plsc.store_scatter(vals_v, [mi * tpw + tok], neg_inf)
            ms.append(m)
            mis.append(mi)

        m0 = ms[0]
        ws = [jnp.exp(m - m0) for m in ms]
        denom = ws[0]
        for w in ws[1:]:
            denom = denom + w
        inv = 1.0 / denom
        for k in range(_K):
            plsc.store_scatter(outp_v, [tok * _E + mis[k]], ws[k] * inv)
            idx_v[pl.ds(k * tpw + col0, _LANES)] = mis[k]
        return c

    lax.fori_loop(0, tpw // _LANES, gbody, 0)

    pltpu.sync_copy(outp_v, outp.at[pl.ds(base * _E, tpw * _E)])
    cps = [
        pltpu.async_copy(
            idx_v.at[pl.ds(k * tpw, tpw)],
            idxT.at[k, pl.ds(base, tpw)], sem)
        for k in range(_K)
    ]
    for cp in cps:
        cp.wait()


def kernel(x_BLD, W_route, b_route, W_noise, b_noise):
    B, L, D = x_BLD.shape
    E = W_route.shape[0]
    N = B * L
    T = 1024
    assert N % T == 0 and E == _E

    info = plsc.get_sparse_core_info()
    nw = info.num_cores * info.num_subcores
    tpw = N // nw
    spb = T // tpw

    x = x_BLD.reshape(N, D)
    wt = jnp.concatenate([W_route, W_noise], axis=0).T
    b = jnp.concatenate([b_route, b_noise]).reshape(1, 2 * E)
    noiseT3 = _noise_const_slabs(B, L, E, tpw)

    noisyT3 = pl.pallas_call(
        functools.partial(_logits_body, tpw),
        grid=(N // T,),
        in_specs=[
            pl.BlockSpec((T, D), lambda i: (i, 0)),
            pl.BlockSpec((D, 2 * E), lambda i: (0, 0)),
            pl.BlockSpec((1, 2 * E), lambda i: (0, 0)),
            pl.BlockSpec((spb, E, tpw), lambda i: (i, 0, 0)),
        ],
        out_specs=pl.BlockSpec((spb, E, tpw), lambda i: (i, 0, 0)),
        out_shape=jax.ShapeDtypeStruct((nw, E, tpw), jnp.float32),
        compiler_params=pltpu.CompilerParams(
            dimension_semantics=("arbitrary",),
        ),
    )(x, wt, b, noiseT3)

    route = functools.partial(
        pl.kernel,
        out_type=[
            jax.ShapeDtypeStruct((N * E,), jnp.float32),
            jax.ShapeDtypeStruct((_K, N), jnp.int32),
        ],
        scratch_types=[
            pltpu.VMEM((E * tpw,), jnp.float32),
            pltpu.VMEM((tpw * E,), jnp.float32),
            pltpu.VMEM((_K * tpw,), jnp.int32),
            pltpu.SemaphoreType.DMA,
        ],
        mesh=plsc.VectorSubcoreMesh(core_axis_name="c", subcore_axis_name="s"),
        compiler_params=pltpu.CompilerParams(needs_layout_passes=False),
    )(functools.partial(_route_body, tpw))

    outp, idxT = route(noisyT3.reshape(nw * E * tpw))
    return outp.reshape(B, L, E), idxT.T.reshape(B, L, _K)
